# baseline (speedup 1.0000x reference)
import numpy as np

import jax
import jax.numpy as jnp
from jax import lax
from jax.experimental import pallas as pl
from jax.experimental.pallas import tpu as pltpu

_BT = 256


def _shift_basis(k, out_size, in_size):
    e = np.zeros((k, out_size, in_size), np.float32)
    for d in range(k):
        for o in range(out_size):
            e[d, o, o + d] = 1.0
    return jnp.asarray(e)


def _lenet_kernel(x_ref, w1b_ref, b1b_ref, w2b_ref,
                  f1w_ref, f1b_ref, f2w_ref, f2b_ref, f3w_ref, f3b_ref,
                  out_ref, xs_ref, p1_ref, p2_ref):
    B = x_ref.shape[-1]
    w1b = w1b_ref[...]
    b1b = b1b_ref[...]
    w2b = w2b_ref[...]

    h = jnp.dot(f1w_ref[...], p2_ref[...],
                preferred_element_type=jnp.float32) + f1b_ref[...]
    h = jnp.maximum(h, 0.0)
    h = jnp.dot(f2w_ref[...], h,
                preferred_element_type=jnp.float32) + f2b_ref[...]
    h = jnp.maximum(h, 0.0)
    out_ref[...] = (jnp.dot(f3w_ref[...], h,
                            preferred_element_type=jnp.float32)
                    + f3b_ref[...])

    @pl.when(pl.program_id(0) == 0)
    def _zero_pad_cols():
        xs_ref[:, 28:32, :] = jnp.zeros((28, 4, B), jnp.bfloat16)

    xs_ref[:, 0:28, :] = x_ref[...]

    for g in range(6):
        win = xs_ref[4 * g:4 * g + 8].reshape(256, B)
        part = jnp.dot(w1b, win,
                       preferred_element_type=jnp.float32) + b1b
        v = part.reshape(4, 144, B)
        for k in range(2):
            r = jnp.maximum(v[2 * k], v[2 * k + 1])
            xm = jnp.maximum(r[0:72], r[72:144])
            p1_ref[2 * g + k, 0:72, :] = (
                jnp.maximum(xm, 0.0).astype(jnp.bfloat16))
    p1_ref[:, 72:80, :] = jnp.ones((12, 8, B), jnp.bfloat16)

    a2 = [jnp.dot(w2b, p1_ref[y:y + 3].reshape(240, B),
                  preferred_element_type=jnp.float32).reshape(10, 16, B)
          for y in range(10)]
    pieces = []
    for yp in range(5):
        t = jnp.maximum(a2[2 * yp], a2[2 * yp + 1])
        pieces.extend(jnp.maximum(t[2 * i], t[2 * i + 1]) for i in range(5))
    p2_ref[...] = jnp.maximum(jnp.concatenate(pieces, axis=0), 0.0)


def kernel(conv1_w, conv1_b, conv2_w, conv2_b, fc1_w, fc1_b,
           fc2_w, fc2_b, fc3_w, fc3_b, x_nchw):
    B = x_nchw.shape[0]
    Bp = ((B + _BT - 1) // _BT) * _BT

    xT = jnp.transpose(x_nchw[:, 0, :, :].astype(jnp.bfloat16), (1, 2, 0))
    if Bp != B:
        xT = jnp.pad(xT, ((0, 0), (0, 0), (0, Bp - B)))

    e1y = _shift_basis(5, 4, 8)
    e1x = _shift_basis(5, 24, 28)
    w1 = conv1_w.astype(jnp.float32).reshape(6, 5, 5)
    t1 = jnp.einsum('oab,ayi->obyi', w1, e1y)
    w1blk = jnp.einsum('obyi,bxj->yoxij', t1, e1x).reshape(576, 224)
    w1blk = jnp.pad(w1blk.reshape(576, 8, 28),
                    ((0, 0), (0, 0), (0, 4))).reshape(576, 256)
    w1blk = w1blk.reshape(4, 6, 12, 2, 256).transpose(0, 3, 1, 2, 4)
    w1blk = w1blk.reshape(576, 256).astype(jnp.bfloat16)
    b1blk = jnp.broadcast_to(conv1_b.astype(jnp.float32)[None, None, :, None],
                             (4, 2, 6, 12)).reshape(576, 1)

    e2x = _shift_basis(3, 10, 12)
    w2 = conv2_w.astype(jnp.float32)
    w2blk = jnp.einsum('ocib,bxj->xoicj', w2, e2x).reshape(160, 3, 72)
    b2rows = jnp.broadcast_to(conv2_b.astype(jnp.float32)[None, :],
                              (10, 16)).reshape(160)
    w2blk = jnp.pad(w2blk, ((0, 0), (0, 0), (0, 8)))
    w2blk = w2blk.at[:, 1, 72].set(b2rows).reshape(160, 240)
    w2blk = w2blk.astype(jnp.bfloat16)

    w1p = jnp.transpose(fc1_w.astype(jnp.float32).reshape(120, 16, 5, 5),
                        (0, 2, 3, 1)).reshape(120, 400)
    b1 = fc1_b.astype(jnp.float32)[:, None]
    w2f = fc2_w.astype(jnp.float32)
    b2 = fc2_b.astype(jnp.float32)[:, None]
    w3f = fc3_w.astype(jnp.float32)
    b3 = fc3_b.astype(jnp.float32)[:, None]

    flops_per_img = 2 * (6 * 576 * 224 + 10 * 160 * 216 + 120 * 400
                         + 84 * 120 + 10 * 84)
    cost = pl.CostEstimate(
        flops=int(flops_per_img * Bp),
        transcendentals=0,
        bytes_accessed=int(4 * Bp * (784 + 10)
                           + 4 * (576 * 224 + 160 * 216 + 120 * 400
                                  + 84 * 120 + 10 * 84)))

    nblk = Bp // _BT
    outT = pl.pallas_call(
        _lenet_kernel,
        out_shape=jax.ShapeDtypeStruct((10, Bp), jnp.float32),
        grid=(nblk + 1,),
        in_specs=[
            pl.BlockSpec((28, 28, _BT),
                         lambda i: (0, 0, jnp.minimum(i, nblk - 1))),
            pl.BlockSpec((576, 256), lambda i: (0, 0)),
            pl.BlockSpec((576, 1), lambda i: (0, 0)),
            pl.BlockSpec((160, 240), lambda i: (0, 0)),
            pl.BlockSpec((120, 400), lambda i: (0, 0)),
            pl.BlockSpec((120, 1), lambda i: (0, 0)),
            pl.BlockSpec((84, 120), lambda i: (0, 0)),
            pl.BlockSpec((84, 1), lambda i: (0, 0)),
            pl.BlockSpec((10, 84), lambda i: (0, 0)),
            pl.BlockSpec((10, 1), lambda i: (0, 0)),
        ],
        out_specs=pl.BlockSpec((10, _BT),
                               lambda i: (0, jnp.maximum(i - 1, 0))),
        scratch_shapes=[
            pltpu.VMEM((28, 32, _BT), jnp.bfloat16),
            pltpu.VMEM((12, 80, _BT), jnp.bfloat16),
            pltpu.VMEM((400, _BT), jnp.float32),
        ],
        compiler_params=pltpu.CompilerParams(
            dimension_semantics=("arbitrary",)),
        cost_estimate=cost,
    )(xT, w1blk, b1blk, w2blk, w1p, b1, w2f, b2, w3f, b3)

    return outT[:, :B].T


# speedup vs baseline: 1.0855x; 1.0855x over previous
import numpy as np

import jax
import jax.numpy as jnp
from jax import lax
from jax.experimental import pallas as pl
from jax.experimental.pallas import tpu as pltpu

_BT = 256


def _shift_basis(k, out_size, in_size):
    e = np.zeros((k, out_size, in_size), np.float32)
    for d in range(k):
        for o in range(out_size):
            e[d, o, o + d] = 1.0
    return jnp.asarray(e)


def _lenet_kernel(x_ref, w1b_ref, b1b_ref, w2b_ref,
                  f1w_ref, f1b_ref, f2w_ref, f2b_ref, f3w_ref, f3b_ref,
                  out_ref, p1_ref, p2_ref):
    B = x_ref.shape[-1]
    w1b = w1b_ref[...]
    b1b = b1b_ref[...]
    w2b = w2b_ref[...]

    h = jnp.dot(f1w_ref[...], p2_ref[...],
                preferred_element_type=jnp.float32) + f1b_ref[...]
    h = jnp.maximum(h, 0.0)
    h = jnp.dot(f2w_ref[...], h,
                preferred_element_type=jnp.float32) + f2b_ref[...]
    h = jnp.maximum(h, 0.0)
    out_ref[...] = (jnp.dot(f3w_ref[...], h,
                            preferred_element_type=jnp.float32)
                    + f3b_ref[...])

    for g in range(6):
        win = x_ref[4 * g:4 * g + 8].reshape(256, B)
        part = jnp.dot(w1b, win,
                       preferred_element_type=jnp.float32) + b1b
        v = part.reshape(4, 144, B)
        for k in range(2):
            r = jnp.maximum(v[2 * k], v[2 * k + 1])
            xm = jnp.maximum(r[0:72], r[72:144])
            p1_ref[2 * g + k, 0:72, :] = (
                jnp.maximum(xm, 0.0).astype(jnp.bfloat16))
    p1_ref[:, 72:80, :] = jnp.ones((12, 8, B), jnp.bfloat16)

    a2 = [jnp.dot(w2b, p1_ref[y:y + 3].reshape(240, B),
                  preferred_element_type=jnp.float32).reshape(10, 16, B)
          for y in range(10)]
    pieces = []
    for yp in range(5):
        t = jnp.maximum(a2[2 * yp], a2[2 * yp + 1])
        pieces.extend(jnp.maximum(t[2 * i], t[2 * i + 1]) for i in range(5))
    p2_ref[...] = jnp.maximum(jnp.concatenate(pieces, axis=0), 0.0)


def kernel(conv1_w, conv1_b, conv2_w, conv2_b, fc1_w, fc1_b,
           fc2_w, fc2_b, fc3_w, fc3_b, x_nchw):
    B = x_nchw.shape[0]
    Bp = ((B + _BT - 1) // _BT) * _BT

    xT = jnp.transpose(x_nchw[:, 0, :, :].astype(jnp.bfloat16), (1, 2, 0))
    xT = jnp.pad(xT, ((0, 0), (0, 4), (0, Bp - B)))

    e1y = _shift_basis(5, 4, 8)
    e1x = _shift_basis(5, 24, 28)
    w1 = conv1_w.astype(jnp.float32).reshape(6, 5, 5)
    t1 = jnp.einsum('oab,ayi->obyi', w1, e1y)
    w1blk = jnp.einsum('obyi,bxj->yoxij', t1, e1x).reshape(576, 224)
    w1blk = jnp.pad(w1blk.reshape(576, 8, 28),
                    ((0, 0), (0, 0), (0, 4))).reshape(576, 256)
    w1blk = w1blk.reshape(4, 6, 12, 2, 256).transpose(0, 3, 1, 2, 4)
    w1blk = w1blk.reshape(576, 256).astype(jnp.bfloat16)
    b1blk = jnp.broadcast_to(conv1_b.astype(jnp.float32)[None, None, :, None],
                             (4, 2, 6, 12)).reshape(576, 1)

    e2x = _shift_basis(3, 10, 12)
    w2 = conv2_w.astype(jnp.float32)
    w2blk = jnp.einsum('ocib,bxj->xoicj', w2, e2x).reshape(160, 3, 72)
    b2rows = jnp.broadcast_to(conv2_b.astype(jnp.float32)[None, :],
                              (10, 16)).reshape(160)
    w2blk = jnp.pad(w2blk, ((0, 0), (0, 0), (0, 8)))
    w2blk = w2blk.at[:, 1, 72].set(b2rows).reshape(160, 240)
    w2blk = w2blk.astype(jnp.bfloat16)

    w1p = jnp.transpose(fc1_w.astype(jnp.float32).reshape(120, 16, 5, 5),
                        (0, 2, 3, 1)).reshape(120, 400)
    b1 = fc1_b.astype(jnp.float32)[:, None]
    w2f = fc2_w.astype(jnp.float32)
    b2 = fc2_b.astype(jnp.float32)[:, None]
    w3f = fc3_w.astype(jnp.float32)
    b3 = fc3_b.astype(jnp.float32)[:, None]

    flops_per_img = 2 * (6 * 576 * 224 + 10 * 160 * 216 + 120 * 400
                         + 84 * 120 + 10 * 84)
    cost = pl.CostEstimate(
        flops=int(flops_per_img * Bp),
        transcendentals=0,
        bytes_accessed=int(4 * Bp * (784 + 10)
                           + 4 * (576 * 224 + 160 * 216 + 120 * 400
                                  + 84 * 120 + 10 * 84)))

    nblk = Bp // _BT
    outT = pl.pallas_call(
        _lenet_kernel,
        out_shape=jax.ShapeDtypeStruct((10, Bp), jnp.float32),
        grid=(nblk + 1,),
        in_specs=[
            pl.BlockSpec((28, 32, _BT),
                         lambda i: (0, 0, jnp.minimum(i, nblk - 1))),
            pl.BlockSpec((576, 256), lambda i: (0, 0)),
            pl.BlockSpec((576, 1), lambda i: (0, 0)),
            pl.BlockSpec((160, 240), lambda i: (0, 0)),
            pl.BlockSpec((120, 400), lambda i: (0, 0)),
            pl.BlockSpec((120, 1), lambda i: (0, 0)),
            pl.BlockSpec((84, 120), lambda i: (0, 0)),
            pl.BlockSpec((84, 1), lambda i: (0, 0)),
            pl.BlockSpec((10, 84), lambda i: (0, 0)),
            pl.BlockSpec((10, 1), lambda i: (0, 0)),
        ],
        out_specs=pl.BlockSpec((10, _BT),
                               lambda i: (0, jnp.maximum(i - 1, 0))),
        scratch_shapes=[
            pltpu.VMEM((12, 80, _BT), jnp.bfloat16),
            pltpu.VMEM((400, _BT), jnp.float32),
        ],
        compiler_params=pltpu.CompilerParams(
            dimension_semantics=("arbitrary",)),
        cost_estimate=cost,
    )(xT, w1blk, b1blk, w2blk, w1p, b1, w2f, b2, w3f, b3)

    return outT[:, :B].T


# speedup vs baseline: 1.1777x; 1.0850x over previous
import numpy as np

import jax
import jax.numpy as jnp
from jax import lax
from jax.experimental import pallas as pl
from jax.experimental.pallas import tpu as pltpu

_BT = 512


def _shift_basis(k, out_size, in_size):
    e = np.zeros((k, out_size, in_size), np.float32)
    for d in range(k):
        for o in range(out_size):
            e[d, o, o + d] = 1.0
    return jnp.asarray(e)


def _lenet_kernel(x_ref, w1b_ref, b1b_ref, w2b_ref,
                  f1w_ref, f1b_ref, f2w_ref, f2b_ref, f3w_ref, f3b_ref,
                  out_ref, p1_ref, p2_ref):
    B = x_ref.shape[-1]
    w1b = w1b_ref[...]
    b1b = b1b_ref[...]
    w2b = w2b_ref[...]

    h = jnp.dot(f1w_ref[...], p2_ref[...],
                preferred_element_type=jnp.float32) + f1b_ref[...]
    h = jnp.maximum(h, 0.0)
    h = jnp.dot(f2w_ref[...], h,
                preferred_element_type=jnp.float32) + f2b_ref[...]
    h = jnp.maximum(h, 0.0)
    out_ref[...] = (jnp.dot(f3w_ref[...], h,
                            preferred_element_type=jnp.float32)
                    + f3b_ref[...])

    for g in range(6):
        win = x_ref[4 * g:4 * g + 8].reshape(256, B)
        part = jnp.dot(w1b, win,
                       preferred_element_type=jnp.float32) + b1b
        v = part.reshape(4, 144, B)
        for k in range(2):
            r = jnp.maximum(v[2 * k], v[2 * k + 1])
            xm = jnp.maximum(r[0:72], r[72:144])
            p1_ref[2 * g + k, 0:72, :] = (
                jnp.maximum(xm, 0.0).astype(jnp.bfloat16))
    p1_ref[:, 72:80, :] = jnp.ones((12, 8, B), jnp.bfloat16)

    a2 = [jnp.dot(w2b, p1_ref[y:y + 3].reshape(240, B),
                  preferred_element_type=jnp.float32).reshape(10, 16, B)
          for y in range(10)]
    pieces = []
    for yp in range(5):
        t = jnp.maximum(a2[2 * yp], a2[2 * yp + 1])
        pieces.extend(jnp.maximum(t[2 * i], t[2 * i + 1]) for i in range(5))
    p2_ref[...] = jnp.maximum(jnp.concatenate(pieces, axis=0), 0.0)


def kernel(conv1_w, conv1_b, conv2_w, conv2_b, fc1_w, fc1_b,
           fc2_w, fc2_b, fc3_w, fc3_b, x_nchw):
    B = x_nchw.shape[0]
    Bp = ((B + _BT - 1) // _BT) * _BT

    xT = jnp.transpose(x_nchw[:, 0, :, :].astype(jnp.bfloat16), (1, 2, 0))
    xT = jnp.pad(xT, ((0, 0), (0, 4), (0, Bp - B)))

    e1y = _shift_basis(5, 4, 8)
    e1x = _shift_basis(5, 24, 28)
    w1 = conv1_w.astype(jnp.float32).reshape(6, 5, 5)
    t1 = jnp.einsum('oab,ayi->obyi', w1, e1y)
    w1blk = jnp.einsum('obyi,bxj->yoxij', t1, e1x).reshape(576, 224)
    w1blk = jnp.pad(w1blk.reshape(576, 8, 28),
                    ((0, 0), (0, 0), (0, 4))).reshape(576, 256)
    w1blk = w1blk.reshape(4, 6, 12, 2, 256).transpose(0, 3, 1, 2, 4)
    w1blk = w1blk.reshape(576, 256).astype(jnp.bfloat16)
    b1blk = jnp.broadcast_to(conv1_b.astype(jnp.float32)[None, None, :, None],
                             (4, 2, 6, 12)).reshape(576, 1)

    e2x = _shift_basis(3, 10, 12)
    w2 = conv2_w.astype(jnp.float32)
    w2blk = jnp.einsum('ocib,bxj->xoicj', w2, e2x).reshape(160, 3, 72)
    b2rows = jnp.broadcast_to(conv2_b.astype(jnp.float32)[None, :],
                              (10, 16)).reshape(160)
    w2blk = jnp.pad(w2blk, ((0, 0), (0, 0), (0, 8)))
    w2blk = w2blk.at[:, 1, 72].set(b2rows).reshape(160, 240)
    w2blk = w2blk.astype(jnp.bfloat16)

    w1p = jnp.transpose(fc1_w.astype(jnp.float32).reshape(120, 16, 5, 5),
                        (0, 2, 3, 1)).reshape(120, 400)
    b1 = fc1_b.astype(jnp.float32)[:, None]
    w2f = fc2_w.astype(jnp.float32)
    b2 = fc2_b.astype(jnp.float32)[:, None]
    w3f = fc3_w.astype(jnp.float32)
    b3 = fc3_b.astype(jnp.float32)[:, None]

    flops_per_img = 2 * (6 * 576 * 224 + 10 * 160 * 216 + 120 * 400
                         + 84 * 120 + 10 * 84)
    cost = pl.CostEstimate(
        flops=int(flops_per_img * Bp),
        transcendentals=0,
        bytes_accessed=int(4 * Bp * (784 + 10)
                           + 4 * (576 * 224 + 160 * 216 + 120 * 400
                                  + 84 * 120 + 10 * 84)))

    nblk = Bp // _BT
    outT = pl.pallas_call(
        _lenet_kernel,
        out_shape=jax.ShapeDtypeStruct((10, Bp), jnp.float32),
        grid=(nblk + 1,),
        in_specs=[
            pl.BlockSpec((28, 32, _BT),
                         lambda i: (0, 0, jnp.minimum(i, nblk - 1))),
            pl.BlockSpec((576, 256), lambda i: (0, 0)),
            pl.BlockSpec((576, 1), lambda i: (0, 0)),
            pl.BlockSpec((160, 240), lambda i: (0, 0)),
            pl.BlockSpec((120, 400), lambda i: (0, 0)),
            pl.BlockSpec((120, 1), lambda i: (0, 0)),
            pl.BlockSpec((84, 120), lambda i: (0, 0)),
            pl.BlockSpec((84, 1), lambda i: (0, 0)),
            pl.BlockSpec((10, 84), lambda i: (0, 0)),
            pl.BlockSpec((10, 1), lambda i: (0, 0)),
        ],
        out_specs=pl.BlockSpec((10, _BT),
                               lambda i: (0, jnp.maximum(i - 1, 0))),
        scratch_shapes=[
            pltpu.VMEM((12, 80, _BT), jnp.bfloat16),
            pltpu.VMEM((400, _BT), jnp.float32),
        ],
        compiler_params=pltpu.CompilerParams(
            dimension_semantics=("arbitrary",)),
        cost_estimate=cost,
    )(xT, w1blk, b1blk, w2blk, w1p, b1, w2f, b2, w3f, b3)

    return outT[:, :B].T


# speedup vs baseline: 1.2173x; 1.0335x over previous
import numpy as np

import jax
import jax.numpy as jnp
from jax import lax
from jax.experimental import pallas as pl
from jax.experimental.pallas import tpu as pltpu

_BT = 1024


def _shift_basis(k, out_size, in_size):
    e = np.zeros((k, out_size, in_size), np.float32)
    for d in range(k):
        for o in range(out_size):
            e[d, o, o + d] = 1.0
    return jnp.asarray(e)


def _lenet_kernel(x_ref, w1b_ref, b1b_ref, w2b_ref,
                  f1w_ref, f1b_ref, f2w_ref, f2b_ref, f3w_ref, f3b_ref,
                  out_ref, p1_ref, p2_ref):
    B = x_ref.shape[-1]
    w1b = w1b_ref[...]
    b1b = b1b_ref[...]
    w2b = w2b_ref[...]

    h = jnp.dot(f1w_ref[...], p2_ref[...],
                preferred_element_type=jnp.float32) + f1b_ref[...]
    h = jnp.maximum(h, 0.0)
    h = jnp.dot(f2w_ref[...], h,
                preferred_element_type=jnp.float32) + f2b_ref[...]
    h = jnp.maximum(h, 0.0)
    out_ref[...] = (jnp.dot(f3w_ref[...], h,
                            preferred_element_type=jnp.float32)
                    + f3b_ref[...])

    for g in range(6):
        win = x_ref[4 * g:4 * g + 8].reshape(256, B)
        part = jnp.dot(w1b, win,
                       preferred_element_type=jnp.float32) + b1b
        v = part.reshape(4, 144, B)
        for k in range(2):
            r = jnp.maximum(v[2 * k], v[2 * k + 1])
            xm = jnp.maximum(r[0:72], r[72:144])
            p1_ref[2 * g + k, 0:72, :] = (
                jnp.maximum(xm, 0.0).astype(jnp.bfloat16))
    p1_ref[:, 72:80, :] = jnp.ones((12, 8, B), jnp.bfloat16)

    a2 = [jnp.dot(w2b, p1_ref[y:y + 3].reshape(240, B),
                  preferred_element_type=jnp.float32).reshape(10, 16, B)
          for y in range(10)]
    pieces = []
    for yp in range(5):
        t = jnp.maximum(a2[2 * yp], a2[2 * yp + 1])
        pieces.extend(jnp.maximum(t[2 * i], t[2 * i + 1]) for i in range(5))
    p2_ref[...] = jnp.maximum(jnp.concatenate(pieces, axis=0), 0.0)


def kernel(conv1_w, conv1_b, conv2_w, conv2_b, fc1_w, fc1_b,
           fc2_w, fc2_b, fc3_w, fc3_b, x_nchw):
    B = x_nchw.shape[0]
    Bp = ((B + _BT - 1) // _BT) * _BT

    xT = jnp.transpose(x_nchw[:, 0, :, :].astype(jnp.bfloat16), (1, 2, 0))
    xT = jnp.pad(xT, ((0, 0), (0, 4), (0, Bp - B)))

    e1y = _shift_basis(5, 4, 8)
    e1x = _shift_basis(5, 24, 28)
    w1 = conv1_w.astype(jnp.float32).reshape(6, 5, 5)
    t1 = jnp.einsum('oab,ayi->obyi', w1, e1y)
    w1blk = jnp.einsum('obyi,bxj->yoxij', t1, e1x).reshape(576, 224)
    w1blk = jnp.pad(w1blk.reshape(576, 8, 28),
                    ((0, 0), (0, 0), (0, 4))).reshape(576, 256)
    w1blk = w1blk.reshape(4, 6, 12, 2, 256).transpose(0, 3, 1, 2, 4)
    w1blk = w1blk.reshape(576, 256).astype(jnp.bfloat16)
    b1blk = jnp.broadcast_to(conv1_b.astype(jnp.float32)[None, None, :, None],
                             (4, 2, 6, 12)).reshape(576, 1)

    e2x = _shift_basis(3, 10, 12)
    w2 = conv2_w.astype(jnp.float32)
    w2blk = jnp.einsum('ocib,bxj->xoicj', w2, e2x).reshape(160, 3, 72)
    b2rows = jnp.broadcast_to(conv2_b.astype(jnp.float32)[None, :],
                              (10, 16)).reshape(160)
    w2blk = jnp.pad(w2blk, ((0, 0), (0, 0), (0, 8)))
    w2blk = w2blk.at[:, 1, 72].set(b2rows).reshape(160, 240)
    w2blk = w2blk.astype(jnp.bfloat16)

    w1p = jnp.transpose(fc1_w.astype(jnp.float32).reshape(120, 16, 5, 5),
                        (0, 2, 3, 1)).reshape(120, 400)
    b1 = fc1_b.astype(jnp.float32)[:, None]
    w2f = fc2_w.astype(jnp.float32)
    b2 = fc2_b.astype(jnp.float32)[:, None]
    w3f = fc3_w.astype(jnp.float32)
    b3 = fc3_b.astype(jnp.float32)[:, None]

    flops_per_img = 2 * (6 * 576 * 224 + 10 * 160 * 216 + 120 * 400
                         + 84 * 120 + 10 * 84)
    cost = pl.CostEstimate(
        flops=int(flops_per_img * Bp),
        transcendentals=0,
        bytes_accessed=int(4 * Bp * (784 + 10)
                           + 4 * (576 * 224 + 160 * 216 + 120 * 400
                                  + 84 * 120 + 10 * 84)))

    nblk = Bp // _BT
    outT = pl.pallas_call(
        _lenet_kernel,
        out_shape=jax.ShapeDtypeStruct((10, Bp), jnp.float32),
        grid=(nblk + 1,),
        in_specs=[
            pl.BlockSpec((28, 32, _BT),
                         lambda i: (0, 0, jnp.minimum(i, nblk - 1))),
            pl.BlockSpec((576, 256), lambda i: (0, 0)),
            pl.BlockSpec((576, 1), lambda i: (0, 0)),
            pl.BlockSpec((160, 240), lambda i: (0, 0)),
            pl.BlockSpec((120, 400), lambda i: (0, 0)),
            pl.BlockSpec((120, 1), lambda i: (0, 0)),
            pl.BlockSpec((84, 120), lambda i: (0, 0)),
            pl.BlockSpec((84, 1), lambda i: (0, 0)),
            pl.BlockSpec((10, 84), lambda i: (0, 0)),
            pl.BlockSpec((10, 1), lambda i: (0, 0)),
        ],
        out_specs=pl.BlockSpec((10, _BT),
                               lambda i: (0, jnp.maximum(i - 1, 0))),
        scratch_shapes=[
            pltpu.VMEM((12, 80, _BT), jnp.bfloat16),
            pltpu.VMEM((400, _BT), jnp.float32),
        ],
        compiler_params=pltpu.CompilerParams(
            dimension_semantics=("arbitrary",)),
        cost_estimate=cost,
    )(xT, w1blk, b1blk, w2blk, w1p, b1, w2f, b2, w3f, b3)

    return outT[:, :B].T
